# baseline (speedup 1.0000x reference)
import jax
import jax.numpy as jnp
from jax import lax
from jax.experimental import pallas as pl
from jax.experimental.pallas import tpu as pltpu
from jax.experimental.pallas import tpu_sc as plsc

N = 10000
E = 320000
G = 64
D = 128
D_OUT = 64
NUM_LAYERS = 5

NC = 2
NS = 16
NW = NC * NS

NP = 10240
EPT = 10240
EP = NW * EPT
CHUNK = 128
STEPS = EPT // CHUNK

import functools


@functools.lru_cache(maxsize=None)
def _mesh():
    return plsc.VectorSubcoreMesh(core_axis_name="c", subcore_axis_name="s",
                                  num_cores=NC, num_subcores=NS)


NBUF = 2
NQ = 2
QR = 2560
TROWS = 168
QACCR = NS * TROWS
DUMMY = QR
SEG = EP // NS
SSTEPS = SEG // CHUNK
PH = 40
CAP = SEG + NBUF * CHUNK
PADDST = 2 * NP


def _deg_body(dst_hbm, out_hbm, idx_all, dacc, sem):
    c = lax.axis_index("c")
    s = lax.axis_index("s")
    wid = c * NS + s
    pltpu.sync_copy(dst_hbm.at[wid], idx_all)

    z = jnp.zeros((16,), jnp.float32)

    def zstep(i, carry):
        dacc[pl.ds(i * 16, 16)] = z
        return carry

    lax.fori_loop(0, NP // 16, zstep, 0)

    ones = jnp.ones((16,), jnp.float32)

    def step(k, carry):
        for j in range(CHUNK // 16):
            idx = idx_all[k, pl.ds(j * 16, 16)]
            plsc.addupdate_scatter(dacc, [idx], ones)
        return carry

    lax.fori_loop(0, STEPS, step, 0)
    pltpu.sync_copy(dacc, out_hbm.at[wid])


@functools.lru_cache(maxsize=None)
def _deg_kernel():
    return pl.kernel(
        _deg_body,
        out_type=jax.ShapeDtypeStruct((NW, NP), jnp.float32),
        mesh=_mesh(),
        compiler_params=pltpu.CompilerParams(needs_layout_passes=False),
        scratch_types=[
            pltpu.VMEM((STEPS, CHUNK), jnp.int32),
            pltpu.VMEM((NP,), jnp.float32),
            pltpu.SemaphoreType.DMA,
        ],
    )


def _agg_body(edges_hbm, table_hbm, zeros_hbm, out_hbm,
              idx_stage, kept_src, kept_dst, rows, sidx, posbuf, acc,
              gsem, ssem):
    c = lax.axis_index("c")
    s = lax.axis_index("s")

    for q in range(NQ):
        pltpu.sync_copy(zeros_hbm, acc.at[pl.ds(s * TROWS, TROWS)])

        base = (c * NQ + q) * QR
        pos0 = jnp.zeros((16,), jnp.int32)

        def phase(ph, pos):
            pltpu.sync_copy(edges_hbm.at[s, :, pl.ds(ph * PH, PH)],
                            idx_stage)

            def prow(k, pos):
                for j in range(CHUNK // 16):
                    srcv = idx_stage[0, k, pl.ds(j * 16, 16)]
                    dstv = idx_stage[1, k, pl.ds(j * 16, 16)]
                    dl = dstv - base
                    mask = (dl >= 0) & (dl < QR)
                    prefix = plsc.cumsum(mask.astype(jnp.int32))
                    positions = pos + prefix - 1
                    plsc.store_scatter(kept_src, [positions], srcv,
                                       mask=mask)
                    plsc.store_scatter(kept_dst, [positions], dl, mask=mask)
                    pos = pos + plsc.all_reduce_population_count(mask)
                return pos

            return lax.fori_loop(0, PH, prow, pos)

        pos = lax.fori_loop(0, SSTEPS // PH, phase, pos0)

        iot = lax.iota(jnp.int32, 16)
        padsrc = jnp.full((16,), N, jnp.int32)
        paddst = jnp.full((16,), DUMMY, jnp.int32)
        for j in range(NBUF * CHUNK // 16):
            plsc.store_scatter(kept_src, [pos + iot + j * 16], padsrc)
            plsc.store_scatter(kept_dst, [pos + iot + j * 16], paddst)
        posbuf[...] = pos
        cnt = posbuf[...][0]
        nblk = (cnt + NBUF * CHUNK - 1) // (NBUF * CHUNK)

        plsc.subcore_barrier()

        def block(t, carry):
            gds = []
            for j in range(NBUF):
                off = (t * NBUF + j) * CHUNK
                gds.append(pltpu.async_copy(
                    table_hbm.at[kept_src.at[pl.ds(off, CHUNK)]],
                    rows.at[j], gsem.at[j]))
            sds = []
            for j in range(NBUF):
                off = (t * NBUF + j) * CHUNK
                for i in range(CHUNK // 16):
                    sidx[j, pl.ds(i * 16, 16)] = \
                        kept_dst[pl.ds(off + i * 16, 16)]
                gds[j].wait()
                sds.append(pltpu.async_copy(
                    rows.at[j], acc.at[sidx.at[j]], ssem.at[j], add=True))
            for d in sds:
                d.wait()
            return carry

        lax.fori_loop(0, nblk, block, 0)
        plsc.subcore_barrier()
        pltpu.sync_copy(acc.at[pl.ds(s * TROWS, TROWS)],
                        out_hbm.at[c, q, pl.ds(s * TROWS, TROWS)])
        if q + 1 < NQ:
            plsc.subcore_barrier()


@functools.lru_cache(maxsize=None)
def _agg_kernel():
    return pl.kernel(
        _agg_body,
        out_type=jax.ShapeDtypeStruct((NC, NQ, QACCR, D), jnp.float32),
        mesh=_mesh(),
        compiler_params=pltpu.CompilerParams(needs_layout_passes=False),
        scratch_types=[
            pltpu.VMEM((2, PH, CHUNK), jnp.int32),
            pltpu.VMEM((CAP,), jnp.int32),
            pltpu.VMEM((CAP,), jnp.int32),
            pltpu.VMEM((NBUF, CHUNK, D), jnp.float32),
            pltpu.VMEM((NBUF, CHUNK), jnp.int32),
            pltpu.VMEM((16,), jnp.int32),
            pltpu.VMEM_SHARED((QACCR, D), jnp.float32),
            pltpu.SemaphoreType.DMA((NBUF,)),
            pltpu.SemaphoreType.DMA((NBUF,)),
        ],
    )


def _first_body(x_ref, w_ref, dp_ref, hpre_ref, hs_ref, dinv_ref):
    dsum = lax.dot_general(dp_ref[...], jnp.ones((NW, 1), jnp.float32),
                           (((0,), (0,)), ((), ())),
                           preferred_element_type=jnp.float32)
    deg = dsum[0:N, :] + 1.0
    dinv = lax.rsqrt(deg)
    dinv_ref[...] = dinv
    h = jnp.dot(x_ref[...], w_ref[...], preferred_element_type=jnp.float32)
    hpre_ref[...] = h
    hs_ref[0:N, :] = h * dinv
    hs_ref[N:NP, :] = jnp.zeros((NP - N, D), jnp.float32)


_first_kernel = pl.pallas_call(
    _first_body,
    out_shape=(
        jax.ShapeDtypeStruct((N, D), jnp.float32),
        jax.ShapeDtypeStruct((NP, D), jnp.float32),
        jax.ShapeDtypeStruct((N, 1), jnp.float32),
    ),
)


def _mid_body(p_ref, hpre_ref, dinv_ref, b_ref, g_ref, be_ref,
              rm_ref, rv_ref, w_ref, hpre_o_ref, hs_o_ref):
    dinv = dinv_ref[...]
    psum = jnp.concatenate(
        [p_ref[i, 0:QR, :] for i in range(NC * NQ - 1)]
        + [p_ref[NC * NQ - 1, 0:N - (NC * NQ - 1) * QR, :]], axis=0)
    agg = psum * dinv + hpre_ref[...] * (dinv * dinv) + b_ref[...]
    o = (agg - rm_ref[...]) * lax.rsqrt(rv_ref[...] + 1e-5) * g_ref[...] \
        + be_ref[...]
    o = jnp.maximum(o, 0.0)
    h = jnp.dot(o, w_ref[...], preferred_element_type=jnp.float32)
    hpre_o_ref[...] = h
    hs_o_ref[0:N, :] = h * dinv
    hs_o_ref[N:NP, :] = jnp.zeros((NP - N, D), jnp.float32)


_mid_kernel = pl.pallas_call(
    _mid_body,
    out_shape=(
        jax.ShapeDtypeStruct((N, D), jnp.float32),
        jax.ShapeDtypeStruct((NP, D), jnp.float32),
    ),
)


def _final_body(p_ref, hpre_ref, dinv_ref, b_ref, batch_ref,
                wout_ref, bout_ref, out_ref):
    dinv = dinv_ref[...]
    psum = jnp.concatenate(
        [p_ref[i, 0:QR, :] for i in range(NC * NQ - 1)]
        + [p_ref[NC * NQ - 1, 0:N - (NC * NQ - 1) * QR, :]], axis=0)
    o = psum * dinv + hpre_ref[...] * (dinv * dinv) + b_ref[...]
    gids = lax.broadcasted_iota(jnp.int32, (1, G), 1)
    onehot = jnp.where(batch_ref[...] == gids, 1.0, 0.0)
    sums = lax.dot_general(onehot, o, (((0,), (0,)), ((), ())),
                           preferred_element_type=jnp.float32)
    counts = lax.dot_general(onehot, jnp.ones((N, 1), jnp.float32),
                             (((0,), (0,)), ((), ())),
                             preferred_element_type=jnp.float32)
    pooled = sums / jnp.maximum(counts, 1.0)
    logits = jnp.dot(pooled, wout_ref[...],
                     preferred_element_type=jnp.float32) + bout_ref[...]
    m = jnp.max(logits, axis=1, keepdims=True)
    z = logits - m
    lse = jnp.log(jnp.sum(jnp.exp(z), axis=1, keepdims=True))
    out_ref[...] = z - lse


_final_kernel = pl.pallas_call(
    _final_body,
    out_shape=jax.ShapeDtypeStruct((G, D_OUT), jnp.float32),
)


def kernel(x, edge_index, batch, Ws, bs, gammas, betas, rms, rvs, Wout, bout):
    pad = jnp.full((EP - E,), N, jnp.int32)
    src_p = jnp.concatenate([edge_index[0].astype(jnp.int32), pad])
    dst_p = jnp.concatenate([edge_index[1].astype(jnp.int32), pad])
    dst_r = dst_p.reshape(NW, STEPS, CHUNK)

    pad_agg = jnp.full((EP - E,), PADDST, jnp.int32)
    dst_a = jnp.concatenate([edge_index[1].astype(jnp.int32), pad_agg])
    edges_ns = jnp.stack([src_p.reshape(NS, SSTEPS, CHUNK),
                          dst_a.reshape(NS, SSTEPS, CHUNK)], axis=1)

    zerosD = jnp.zeros((TROWS, D), jnp.float32)

    degp = _deg_kernel()(dst_r)

    hpre, hs, dinv = _first_kernel(x.astype(jnp.float32), Ws[0], degp)

    for i in range(1, NUM_LAYERS):
        p = _agg_kernel()(edges_ns, hs, zerosD)
        j = i - 1
        hpre, hs = _mid_kernel(
            p.reshape(NC * NQ, QACCR, D), hpre, dinv,
            bs[j].reshape(1, D), gammas[j].reshape(1, D),
            betas[j].reshape(1, D), rms[j].reshape(1, D),
            rvs[j].reshape(1, D), Ws[i])

    p = _agg_kernel()(edges_ns, hs, zerosD)
    return _final_kernel(p.reshape(NC * NQ, QACCR, D), hpre, dinv,
                         bs[NUM_LAYERS - 1].reshape(1, D),
                         batch.reshape(N, 1).astype(jnp.int32),
                         Wout, bout.reshape(1, D_OUT))


# speedup vs baseline: 1.5889x; 1.5889x over previous
import jax
import jax.numpy as jnp
from jax import lax
from jax.experimental import pallas as pl
from jax.experimental.pallas import tpu as pltpu
from jax.experimental.pallas import tpu_sc as plsc

N = 10000
E = 320000
G = 64
D = 128
D_OUT = 64
NUM_LAYERS = 5

NC = 2
NS = 16
NW = NC * NS

NP = 10240
EPT = 10240
EP = NW * EPT
CHUNK = 128
STEPS = EPT // CHUNK

import functools


@functools.lru_cache(maxsize=None)
def _mesh():
    return plsc.VectorSubcoreMesh(core_axis_name="c", subcore_axis_name="s",
                                  num_cores=NC, num_subcores=NS)


NBUF = 2
NQ = 1
QR = 5120
TROWS = 328
QACCR = NS * TROWS
DUMMY = QR
SEG = EP // NS
SSTEPS = SEG // CHUNK
PH = 40
CAP = SEG + NBUF * CHUNK
PADDST = 2 * NP


def _deg_body(dst_hbm, out_hbm, idx_all, dacc, sem):
    c = lax.axis_index("c")
    s = lax.axis_index("s")
    wid = c * NS + s
    pltpu.sync_copy(dst_hbm.at[wid], idx_all)

    z = jnp.zeros((16,), jnp.float32)

    def zstep(i, carry):
        dacc[pl.ds(i * 16, 16)] = z
        return carry

    lax.fori_loop(0, NP // 16, zstep, 0)

    ones = jnp.ones((16,), jnp.float32)

    def step(k, carry):
        for j in range(CHUNK // 16):
            idx = idx_all[k, pl.ds(j * 16, 16)]
            plsc.addupdate_scatter(dacc, [idx], ones)
        return carry

    lax.fori_loop(0, STEPS, step, 0)
    pltpu.sync_copy(dacc, out_hbm.at[wid])


@functools.lru_cache(maxsize=None)
def _deg_kernel():
    return pl.kernel(
        _deg_body,
        out_type=jax.ShapeDtypeStruct((NW, NP), jnp.float32),
        mesh=_mesh(),
        compiler_params=pltpu.CompilerParams(needs_layout_passes=False),
        scratch_types=[
            pltpu.VMEM((STEPS, CHUNK), jnp.int32),
            pltpu.VMEM((NP,), jnp.float32),
            pltpu.SemaphoreType.DMA,
        ],
    )


def _compact_body(edges_hbm, kept_hbm, cnt_hbm,
                  idx_stage, kept_src, kept_dst, posbuf, sem):
    c = lax.axis_index("c")
    s = lax.axis_index("s")
    base = c * QR
    pos0 = jnp.zeros((16,), jnp.int32)

    def phase(ph, pos):
        pltpu.sync_copy(edges_hbm.at[s, :, pl.ds(ph * PH, PH)], idx_stage)

        def prow(k, pos):
            for j in range(CHUNK // 16):
                srcv = idx_stage[0, k, pl.ds(j * 16, 16)]
                dstv = idx_stage[1, k, pl.ds(j * 16, 16)]
                dl = dstv - base
                mask = (dl >= 0) & (dl < QR)
                prefix = plsc.cumsum(mask.astype(jnp.int32))
                positions = pos + prefix - 1
                plsc.store_scatter(kept_src, [positions], srcv, mask=mask)
                plsc.store_scatter(kept_dst, [positions], dl, mask=mask)
                pos = pos + plsc.all_reduce_population_count(mask)
            return pos

        return lax.fori_loop(0, PH, prow, pos)

    pos = lax.fori_loop(0, SSTEPS // PH, phase, pos0)

    iot = lax.iota(jnp.int32, 16)
    padsrc = jnp.full((16,), N, jnp.int32)
    paddst = jnp.full((16,), DUMMY, jnp.int32)
    for j in range(NBUF * CHUNK // 16):
        plsc.store_scatter(kept_src, [pos + iot + j * 16], padsrc)
        plsc.store_scatter(kept_dst, [pos + iot + j * 16], paddst)
    posbuf[...] = pos
    pltpu.sync_copy(kept_src, kept_hbm.at[c, s, 0])
    pltpu.sync_copy(kept_dst, kept_hbm.at[c, s, 1])
    pltpu.sync_copy(posbuf, cnt_hbm.at[c, s])


@functools.lru_cache(maxsize=None)
def _compact_kernel():
    return pl.kernel(
        _compact_body,
        out_type=(
            jax.ShapeDtypeStruct((NC, NS, 2, CAP), jnp.int32),
            jax.ShapeDtypeStruct((NC, NS, 16), jnp.int32),
        ),
        mesh=_mesh(),
        compiler_params=pltpu.CompilerParams(needs_layout_passes=False),
        scratch_types=[
            pltpu.VMEM((2, PH, CHUNK), jnp.int32),
            pltpu.VMEM((CAP,), jnp.int32),
            pltpu.VMEM((CAP,), jnp.int32),
            pltpu.VMEM((16,), jnp.int32),
            pltpu.SemaphoreType.DMA,
        ],
    )


def _agg_body(kept_hbm, cnt_hbm, table_hbm, zeros_hbm, out_hbm,
              kept_src, kept_dst, rows, sidx, posbuf, acc, gsem, ssem):
    c = lax.axis_index("c")
    s = lax.axis_index("s")
    pltpu.sync_copy(zeros_hbm, acc.at[pl.ds(s * TROWS, TROWS)])
    pltpu.sync_copy(kept_hbm.at[c, s, 0], kept_src)
    pltpu.sync_copy(kept_hbm.at[c, s, 1], kept_dst)
    pltpu.sync_copy(cnt_hbm.at[c, s], posbuf)
    cnt = posbuf[...][0]
    nblk = (cnt + NBUF * CHUNK - 1) // (NBUF * CHUNK)

    plsc.subcore_barrier()

    def block(t, carry):
        gds = []
        for j in range(NBUF):
            off = (t * NBUF + j) * CHUNK
            gds.append(pltpu.async_copy(
                table_hbm.at[kept_src.at[pl.ds(off, CHUNK)]],
                rows.at[j], gsem.at[j]))
        sds = []
        for j in range(NBUF):
            off = (t * NBUF + j) * CHUNK
            for i in range(CHUNK // 16):
                sidx[j, pl.ds(i * 16, 16)] = kept_dst[pl.ds(off + i * 16, 16)]
            gds[j].wait()
            sds.append(pltpu.async_copy(
                rows.at[j], acc.at[sidx.at[j]], ssem.at[j], add=True))
        for d in sds:
            d.wait()
        return carry

    lax.fori_loop(0, nblk, block, 0)
    plsc.subcore_barrier()
    pltpu.sync_copy(acc.at[pl.ds(s * TROWS, TROWS)],
                    out_hbm.at[c, 0, pl.ds(s * TROWS, TROWS)])


@functools.lru_cache(maxsize=None)
def _agg_kernel():
    return pl.kernel(
        _agg_body,
        out_type=jax.ShapeDtypeStruct((NC, NQ, QACCR, D), jnp.float32),
        mesh=_mesh(),
        compiler_params=pltpu.CompilerParams(needs_layout_passes=False),
        scratch_types=[
            pltpu.VMEM((CAP,), jnp.int32),
            pltpu.VMEM((CAP,), jnp.int32),
            pltpu.VMEM((NBUF, CHUNK, D), jnp.float32),
            pltpu.VMEM((NBUF, CHUNK), jnp.int32),
            pltpu.VMEM((16,), jnp.int32),
            pltpu.VMEM_SHARED((QACCR, D), jnp.float32),
            pltpu.SemaphoreType.DMA((NBUF,)),
            pltpu.SemaphoreType.DMA((NBUF,)),
        ],
    )


def _first_body(x_ref, w_ref, dp_ref, hpre_ref, hs_ref, dinv_ref):
    dsum = lax.dot_general(dp_ref[...], jnp.ones((NW, 1), jnp.float32),
                           (((0,), (0,)), ((), ())),
                           preferred_element_type=jnp.float32)
    deg = dsum[0:N, :] + 1.0
    dinv = lax.rsqrt(deg)
    dinv_ref[...] = dinv
    h = jnp.dot(x_ref[...], w_ref[...], preferred_element_type=jnp.float32)
    hpre_ref[...] = h
    hs_ref[0:N, :] = h * dinv
    hs_ref[N:NP, :] = jnp.zeros((NP - N, D), jnp.float32)


_first_kernel = pl.pallas_call(
    _first_body,
    out_shape=(
        jax.ShapeDtypeStruct((N, D), jnp.float32),
        jax.ShapeDtypeStruct((NP, D), jnp.float32),
        jax.ShapeDtypeStruct((N, 1), jnp.float32),
    ),
)


def _mid_body(p_ref, hpre_ref, dinv_ref, b_ref, g_ref, be_ref,
              rm_ref, rv_ref, w_ref, hpre_o_ref, hs_o_ref):
    dinv = dinv_ref[...]
    psum = jnp.concatenate(
        [p_ref[i, 0:QR, :] for i in range(NC * NQ - 1)]
        + [p_ref[NC * NQ - 1, 0:N - (NC * NQ - 1) * QR, :]], axis=0)
    agg = psum * dinv + hpre_ref[...] * (dinv * dinv) + b_ref[...]
    o = (agg - rm_ref[...]) * lax.rsqrt(rv_ref[...] + 1e-5) * g_ref[...] \
        + be_ref[...]
    o = jnp.maximum(o, 0.0)
    h = jnp.dot(o, w_ref[...], preferred_element_type=jnp.float32)
    hpre_o_ref[...] = h
    hs_o_ref[0:N, :] = h * dinv
    hs_o_ref[N:NP, :] = jnp.zeros((NP - N, D), jnp.float32)


_mid_kernel = pl.pallas_call(
    _mid_body,
    out_shape=(
        jax.ShapeDtypeStruct((N, D), jnp.float32),
        jax.ShapeDtypeStruct((NP, D), jnp.float32),
    ),
)


def _final_body(p_ref, hpre_ref, dinv_ref, b_ref, batch_ref,
                wout_ref, bout_ref, out_ref):
    dinv = dinv_ref[...]
    psum = jnp.concatenate(
        [p_ref[i, 0:QR, :] for i in range(NC * NQ - 1)]
        + [p_ref[NC * NQ - 1, 0:N - (NC * NQ - 1) * QR, :]], axis=0)
    o = psum * dinv + hpre_ref[...] * (dinv * dinv) + b_ref[...]
    gids = lax.broadcasted_iota(jnp.int32, (1, G), 1)
    onehot = jnp.where(batch_ref[...] == gids, 1.0, 0.0)
    sums = lax.dot_general(onehot, o, (((0,), (0,)), ((), ())),
                           preferred_element_type=jnp.float32)
    counts = lax.dot_general(onehot, jnp.ones((N, 1), jnp.float32),
                             (((0,), (0,)), ((), ())),
                             preferred_element_type=jnp.float32)
    pooled = sums / jnp.maximum(counts, 1.0)
    logits = jnp.dot(pooled, wout_ref[...],
                     preferred_element_type=jnp.float32) + bout_ref[...]
    m = jnp.max(logits, axis=1, keepdims=True)
    z = logits - m
    lse = jnp.log(jnp.sum(jnp.exp(z), axis=1, keepdims=True))
    out_ref[...] = z - lse


_final_kernel = pl.pallas_call(
    _final_body,
    out_shape=jax.ShapeDtypeStruct((G, D_OUT), jnp.float32),
)


def kernel(x, edge_index, batch, Ws, bs, gammas, betas, rms, rvs, Wout, bout):
    pad = jnp.full((EP - E,), N, jnp.int32)
    src_p = jnp.concatenate([edge_index[0].astype(jnp.int32), pad])
    dst_p = jnp.concatenate([edge_index[1].astype(jnp.int32), pad])
    dst_r = dst_p.reshape(NW, STEPS, CHUNK)

    pad_agg = jnp.full((EP - E,), PADDST, jnp.int32)
    dst_a = jnp.concatenate([edge_index[1].astype(jnp.int32), pad_agg])
    edges_ns = jnp.stack([src_p.reshape(NS, SSTEPS, CHUNK),
                          dst_a.reshape(NS, SSTEPS, CHUNK)], axis=1)

    zerosD = jnp.zeros((TROWS, D), jnp.float32)

    degp = _deg_kernel()(dst_r)
    kept, cnts = _compact_kernel()(edges_ns)

    hpre, hs, dinv = _first_kernel(x.astype(jnp.float32), Ws[0], degp)

    for i in range(1, NUM_LAYERS):
        p = _agg_kernel()(kept, cnts, hs, zerosD)
        j = i - 1
        hpre, hs = _mid_kernel(
            p.reshape(NC * NQ, QACCR, D), hpre, dinv,
            bs[j].reshape(1, D), gammas[j].reshape(1, D),
            betas[j].reshape(1, D), rms[j].reshape(1, D),
            rvs[j].reshape(1, D), Ws[i])

    p = _agg_kernel()(kept, cnts, hs, zerosD)
    return _final_kernel(p.reshape(NC * NQ, QACCR, D), hpre, dinv,
                         bs[NUM_LAYERS - 1].reshape(1, D),
                         batch.reshape(N, 1).astype(jnp.int32),
                         Wout, bout.reshape(1, D_OUT))
